# baseline (speedup 1.0000x reference)
import functools

import jax
import jax.numpy as jnp
import numpy as np
from jax.experimental import pallas as pl
from jax.experimental.pallas import tpu as pltpu
from jax.sharding import Mesh, PartitionSpec as P

_B, _C, _H, _W = 4, 512, 128, 128
_K = 171
_IGNORE = 255
_M = 1024


def _loss_block_kernel(f_ref, lab_ref, p_ref, t_ref, loss_ref, w_ref):
    f = f_ref[0]
    lab = lab_ref[0]
    p = p_ref[...]
    t = t_ref[...]

    nrm2 = jnp.sum(f * f, axis=0, keepdims=True)
    inv_norm = jax.lax.rsqrt(jnp.maximum(nrm2, 1e-24))

    s = jax.lax.dot_general(
        p, f, (((1,), (0,)), ((), ())),
        preferred_element_type=jnp.float32,
        precision=jax.lax.Precision.HIGHEST,
    )

    inv_t = 1.0 / jnp.clip(t, 0.01, 1.0)
    logits = s * inv_norm * inv_t.T

    mx = jnp.max(logits, axis=0, keepdims=True)
    lse = jnp.log(jnp.sum(jnp.exp(logits - mx), axis=0, keepdims=True)) + mx

    safe_lab = jnp.clip(lab, 0, _K - 1)
    kiota = jax.lax.broadcasted_iota(jnp.int32, logits.shape, 0)
    picked = jnp.sum(jnp.where(kiota == safe_lab, logits, 0.0),
                     axis=0, keepdims=True)

    valid = (lab != _IGNORE).astype(jnp.float32)
    loss_ref[0] = jnp.sum((lse - picked) * valid, axis=1, keepdims=True)
    w_ref[0] = jnp.sum(valid, axis=1, keepdims=True)


def _shard_sums(features, labels, prototypes, class_temperature):
    b, c, h, w = features.shape
    k = prototypes.shape[0]
    hw = h * w
    nm = hw // _M
    grid = b * nm

    feats = features.reshape(b, c, hw)
    labs = labels.reshape(grid, 1, _M)
    temps = class_temperature.reshape(1, k)

    loss_parts, w_parts = pl.pallas_call(
        _loss_block_kernel,
        grid=(grid,),
        in_specs=[
            pl.BlockSpec((1, c, _M), lambda i: (i // nm, 0, i % nm)),
            pl.BlockSpec((1, 1, _M), lambda i: (i, 0, 0)),
            pl.BlockSpec((k, c), lambda i: (0, 0)),
            pl.BlockSpec((1, k), lambda i: (0, 0)),
        ],
        out_specs=[
            pl.BlockSpec((1, 1, 1), lambda i: (i, 0, 0)),
            pl.BlockSpec((1, 1, 1), lambda i: (i, 0, 0)),
        ],
        out_shape=[
            jax.ShapeDtypeStruct((grid, 1, 1), jnp.float32),
            jax.ShapeDtypeStruct((grid, 1, 1), jnp.float32),
        ],
        compiler_params=pltpu.CompilerParams(
            dimension_semantics=("parallel",),
        ),
    )(feats, labs, prototypes, temps)

    return jnp.sum(loss_parts), jnp.sum(w_parts)


@functools.partial(jax.jit, static_argnames=())
def kernel(features, labels, prototypes, class_temperature):
    b = features.shape[0]
    devs = jax.devices()
    nd = 1
    for cand_nd in (4, 2):
        if len(devs) >= cand_nd and b % cand_nd == 0:
            nd = cand_nd
            break

    if nd == 1:
        loss_sum, w_sum = _shard_sums(features, labels, prototypes,
                                      class_temperature)
        return loss_sum / jnp.maximum(w_sum, 1.0)

    mesh = Mesh(np.array(devs[:nd]), ("d",))

    def per_shard(f, l, p, t):
        loss_sum, w_sum = _shard_sums(f, l, p, t)
        loss_sum = jax.lax.psum(loss_sum, "d")
        w_sum = jax.lax.psum(w_sum, "d")
        return loss_sum / jnp.maximum(w_sum, 1.0)

    fn = jax.shard_map(
        per_shard,
        mesh=mesh,
        in_specs=(P("d"), P("d"), P(), P()),
        out_specs=P(),
        check_vma=False,
    )
    return fn(features, labels, prototypes, class_temperature)


# speedup vs baseline: 2.7454x; 2.7454x over previous
import functools

import jax
import jax.numpy as jnp
import numpy as np
from jax.experimental import pallas as pl
from jax.experimental.pallas import tpu as pltpu
from jax.sharding import Mesh, PartitionSpec as P

_B, _C, _H, _W = 4, 512, 128, 128
_K = 171
_IGNORE = 255
_M = 4096


def _loss_block_kernel(f_ref, lab_ref, p_ref, t_ref, loss_ref, w_ref):
    f = f_ref[0]
    lab = lab_ref[0]
    p = p_ref[...]
    t = t_ref[...]

    nrm2 = jnp.sum(f * f, axis=0, keepdims=True)
    inv_norm = jax.lax.rsqrt(jnp.maximum(nrm2, 1e-24))

    s = jax.lax.dot_general(
        p, f, (((1,), (0,)), ((), ())),
        preferred_element_type=jnp.float32,
        precision=jax.lax.Precision.HIGHEST,
    )

    inv_t = 1.0 / jnp.clip(t, 0.01, 1.0)
    logits = s * inv_norm * inv_t.T

    mx = jnp.max(logits, axis=0, keepdims=True)
    lse = jnp.log(jnp.sum(jnp.exp(logits - mx), axis=0, keepdims=True)) + mx

    safe_lab = jnp.clip(lab, 0, _K - 1)
    kiota = jax.lax.broadcasted_iota(jnp.int32, logits.shape, 0)
    picked = jnp.sum(jnp.where(kiota == safe_lab, logits, 0.0),
                     axis=0, keepdims=True)

    valid = (lab != _IGNORE).astype(jnp.float32)
    loss_ref[0] = jnp.sum((lse - picked) * valid, axis=1, keepdims=True)
    w_ref[0] = jnp.sum(valid, axis=1, keepdims=True)


def _shard_sums(features, labels, prototypes, class_temperature):
    b, c, h, w = features.shape
    k = prototypes.shape[0]
    hw = h * w
    nm = hw // _M
    grid = b * nm

    feats = features.reshape(b, c, hw)
    labs = labels.reshape(grid, 1, _M)
    temps = class_temperature.reshape(1, k)

    loss_parts, w_parts = pl.pallas_call(
        _loss_block_kernel,
        grid=(grid,),
        in_specs=[
            pl.BlockSpec((1, c, _M), lambda i: (i // nm, 0, i % nm)),
            pl.BlockSpec((1, 1, _M), lambda i: (i, 0, 0)),
            pl.BlockSpec((k, c), lambda i: (0, 0)),
            pl.BlockSpec((1, k), lambda i: (0, 0)),
        ],
        out_specs=[
            pl.BlockSpec((1, 1, 1), lambda i: (i, 0, 0)),
            pl.BlockSpec((1, 1, 1), lambda i: (i, 0, 0)),
        ],
        out_shape=[
            jax.ShapeDtypeStruct((grid, 1, 1), jnp.float32),
            jax.ShapeDtypeStruct((grid, 1, 1), jnp.float32),
        ],
        compiler_params=pltpu.CompilerParams(
            dimension_semantics=("parallel",),
        ),
    )(feats, labs, prototypes, temps)

    return jnp.sum(loss_parts), jnp.sum(w_parts)


@functools.partial(jax.jit, static_argnames=())
def kernel(features, labels, prototypes, class_temperature):
    loss_sum, w_sum = _shard_sums(features, labels, prototypes,
                                  class_temperature)
    return loss_sum / jnp.maximum(w_sum, 1.0)
